# baseline (speedup 1.0000x reference)
import functools

import jax
import jax.numpy as jnp
from jax import lax
from jax.experimental import pallas as pl
from jax.experimental.pallas import tpu as pltpu
from jax.experimental.pallas import tpu_sc as plsc

V = 100000
D = 128
Q = 1024
K5 = 5
VT = 2048
NT = 49
CW = 128
CPT = VT // CW
NCH = NT * CPT
NCAND = K5 * CW
_REAL_LAST = V - (NT - 1) * VT
_FULL_SLABS = _REAL_LAST // CW
_REM = _REAL_LAST % CW
IMAX = jnp.iinfo(jnp.int32).max


@functools.lru_cache(maxsize=1)
def _make_sc_query_gather():
    info = plsc.get_sparse_core_info()
    nw = info.num_cores * info.num_subcores
    b_per_w = Q // nw
    mesh = plsc.VectorSubcoreMesh(core_axis_name="c", subcore_axis_name="s")

    @functools.partial(
        pl.kernel, mesh=mesh,
        out_type=jax.ShapeDtypeStruct((Q, D), jnp.float32),
        scratch_types=[
            pltpu.VMEM((b_per_w,), jnp.int32),
            pltpu.VMEM((b_per_w, D), jnp.float32),
            pltpu.SemaphoreType.DMA,
        ],
    )
    def gather_kernel(table_hbm, idx_hbm, out_hbm, idx_v, rows_v, sem):
        wid = lax.axis_index("s") * info.num_cores + lax.axis_index("c")
        base = wid * b_per_w
        pltpu.sync_copy(idx_hbm.at[pl.ds(base, b_per_w)], idx_v)
        pltpu.async_copy(table_hbm.at[idx_v], rows_v, sem).wait()
        pltpu.sync_copy(rows_v, out_hbm.at[pl.ds(base, b_per_w)])

    return gather_kernel


@functools.lru_cache(maxsize=1)
def _make_sc_chunk_gather():
    info = plsc.get_sparse_core_info()
    nw = info.num_cores * info.num_subcores
    b_q = Q // nw
    mesh = plsc.VectorSubcoreMesh(core_axis_name="c", subcore_axis_name="s")

    @functools.partial(
        pl.kernel, mesh=mesh,
        out_type=jax.ShapeDtypeStruct((K5 * Q, CW), jnp.float32),
        scratch_types=[
            pltpu.VMEM((K5, b_q), jnp.int32),
            pltpu.VMEM((K5 * b_q, CW), jnp.float32),
            pltpu.SemaphoreType.DMA,
        ],
    )
    def gather_kernel(table_hbm, i0, i1, i2, i3, i4, out_hbm,
                      idx_v, rows_v, sem):
        wid = lax.axis_index("s") * info.num_cores + lax.axis_index("c")
        base = wid * b_q
        idx_hbms = [i0, i1, i2, i3, i4]
        for s in range(K5):
            pltpu.sync_copy(idx_hbms[s].at[pl.ds(base, b_q)], idx_v.at[s])
        copies = [
            pltpu.async_copy(table_hbm.at[idx_v.at[s]],
                             rows_v.at[pl.ds(s * b_q, b_q)], sem)
            for s in range(K5)
        ]
        for c in copies:
            c.wait()
        for s in range(K5):
            pltpu.sync_copy(rows_v.at[pl.ds(s * b_q, b_q)],
                            out_hbm.at[pl.ds(s * Q + base, b_q)])

    return gather_kernel


def _extract_topk(vals, ids, n):
    out_v, out_i = [], []
    for r in range(n):
        m = jnp.max(vals, axis=1, keepdims=True)
        sel = jnp.min(jnp.where(vals == m, ids, IMAX), axis=1, keepdims=True)
        out_v.append(m)
        out_i.append(sel)
        if r < n - 1:
            vals = jnp.where(ids == sel, -jnp.inf, vals)
    return jnp.concatenate(out_v, axis=1), jnp.concatenate(out_i, axis=1)


def _extract_topk_ax0(vals, ids, n):
    out_v, out_i = [], []
    for r in range(n):
        m = jnp.max(vals, axis=0, keepdims=True)
        sel = jnp.min(jnp.where(vals == m, ids, IMAX), axis=0, keepdims=True)
        out_v.append(m)
        out_i.append(sel)
        if r < n - 1:
            vals = jnp.where(ids == sel, -jnp.inf, vals)
    return jnp.concatenate(out_v, axis=0), jnp.concatenate(out_i, axis=0)


def _body_a(q_ref, emb_ref, sims_ref, base_ref,
            i0_ref, i1_ref, i2_ref, i3_ref, i4_ref, qn_ref, cmax_ref):
    v = pl.program_id(0)

    @pl.when(v == 0)
    def _init():
        q = q_ref[...]
        qn_ref[...] = q / jnp.sqrt(jnp.sum(q * q, axis=1, keepdims=True))

    qn = qn_ref[...]
    parts = []
    slabs = []
    for s in range(CPT):
        emb_s = emb_ref[pl.ds(s * CW, CW), :]
        embn_s = emb_s / jnp.sqrt(jnp.sum(emb_s * emb_s, axis=1,
                                          keepdims=True))
        slab = lax.dot_general(qn, embn_s, (((1,), (1,)), ((), ())),
                               preferred_element_type=jnp.float32)
        slabs.append(slab)
        sims_ref[s] = slab
        parts.append(jnp.max(slab, axis=1, keepdims=True))
    cmax_ref[pl.ds(pl.multiple_of(v * CPT, CPT), CPT), :] = (
        jnp.concatenate(parts, axis=1).T)

    @pl.when(v == NT - 1)
    def _fix_tail_and_select():
        fparts = []
        if _REM:
            lane = lax.broadcasted_iota(jnp.int32, (Q, CW), 1)
            s_p = slabs[_FULL_SLABS]
            fparts.append(jnp.max(jnp.where(lane < _REM, s_p, -jnp.inf),
                                  axis=1, keepdims=True))
        while _FULL_SLABS + len(fparts) < CPT:
            fparts.append(jnp.full((Q, 1), -jnp.inf, jnp.float32))
        cmax_ref[pl.ds(NCH - len(fparts), len(fparts)), :] = (
            jnp.concatenate(fparts, axis=1).T)

        cids = lax.broadcasted_iota(jnp.int32, (NCH, Q), 0)
        _, top_c = _extract_topk_ax0(cmax_ref[...], cids, K5)
        base_ref[...] = (top_c * CW).T
        flat = top_c * Q + lax.broadcasted_iota(jnp.int32, (K5, Q), 1)
        for s, ref in enumerate([i0_ref, i1_ref, i2_ref, i3_ref, i4_ref]):
            ref[...] = flat[s]


_tc_a = pl.pallas_call(
    _body_a,
    grid=(NT,),
    in_specs=[
        pl.BlockSpec((Q, D), lambda v: (0, 0)),
        pl.BlockSpec((VT, D), lambda v: (v, 0)),
    ],
    out_specs=[
        pl.BlockSpec((CPT, Q, CW), lambda v: (v, 0, 0)),
        pl.BlockSpec((Q, K5), lambda v: (0, 0)),
    ] + [pl.BlockSpec((Q,), lambda v: (0,)) for _ in range(K5)],
    out_shape=[
        jax.ShapeDtypeStruct((NCH, Q, CW), jnp.float32),
        jax.ShapeDtypeStruct((Q, K5), jnp.int32),
    ] + [jax.ShapeDtypeStruct((Q,), jnp.int32) for _ in range(K5)],
    scratch_shapes=[
        pltpu.VMEM((Q, D), jnp.float32),
        pltpu.VMEM((NCH, Q), jnp.float32),
    ],
    compiler_params=pltpu.CompilerParams(
        dimension_semantics=("arbitrary",)),
)


def _body_b(k_ref, cand_ref, base_ref, outv_ref, outi_ref):
    cand = jnp.concatenate([cand_ref[s] for s in range(K5)], axis=1)
    cb = base_ref[...]
    iota = lax.broadcasted_iota(jnp.int32, (Q, CW), 1)
    ids = jnp.concatenate([cb[:, s:s + 1] + iota for s in range(K5)], axis=1)
    cand = jnp.where(ids < V, cand, -jnp.inf)
    top_v, top_i = _extract_topk(cand, ids, K5)
    outv_ref[...] = top_v
    outi_ref[...] = top_i + (k_ref[0] - K5)


_tc_b = pl.pallas_call(
    _body_b,
    in_specs=[
        pl.BlockSpec(memory_space=pltpu.SMEM),
        pl.BlockSpec(),
        pl.BlockSpec(),
    ],
    out_shape=[
        jax.ShapeDtypeStruct((Q, K5), jnp.float32),
        jax.ShapeDtypeStruct((Q, K5), jnp.int32),
    ],
)


def kernel(embedding, query_ids, k):
    q = _make_sc_query_gather()(embedding, query_ids.astype(jnp.int32))
    sims, col_base, i0, i1, i2, i3, i4 = _tc_a(q, embedding)
    table = sims.reshape(NCH * Q, CW)
    gathered = _make_sc_chunk_gather()(table, i0, i1, i2, i3, i4)
    cand3 = gathered.reshape(K5, Q, CW)
    k_arr = jnp.asarray(k, jnp.int32).reshape(1)
    top_v, top_i = _tc_b(k_arr, cand3, col_base)
    return top_v, top_i


# speedup vs baseline: 1.6943x; 1.6943x over previous
import functools

import jax
import jax.numpy as jnp
from jax import lax
from jax.experimental import pallas as pl
from jax.experimental.pallas import tpu as pltpu
from jax.experimental.pallas import tpu_sc as plsc

V = 100000
D = 128
Q = 1024
K5 = 5
VT = 4096
NT = 25
CW = 128
CPT = VT // CW
NCH = NT * CPT
NCAND = K5 * CW
_REAL_LAST = V - (NT - 1) * VT
_FULL_SLABS = _REAL_LAST // CW
_REM = _REAL_LAST % CW
IMAX = jnp.iinfo(jnp.int32).max


@functools.lru_cache(maxsize=1)
def _make_sc_query_gather():
    info = plsc.get_sparse_core_info()
    nw = info.num_cores * info.num_subcores
    b_per_w = Q // nw
    mesh = plsc.VectorSubcoreMesh(core_axis_name="c", subcore_axis_name="s")

    @functools.partial(
        pl.kernel, mesh=mesh,
        out_type=jax.ShapeDtypeStruct((Q, D), jnp.float32),
        scratch_types=[
            pltpu.VMEM((b_per_w,), jnp.int32),
            pltpu.VMEM((b_per_w, D), jnp.float32),
            pltpu.SemaphoreType.DMA,
        ],
    )
    def gather_kernel(table_hbm, idx_hbm, out_hbm, idx_v, rows_v, sem):
        wid = lax.axis_index("s") * info.num_cores + lax.axis_index("c")
        base = wid * b_per_w
        pltpu.sync_copy(idx_hbm.at[pl.ds(base, b_per_w)], idx_v)
        pltpu.async_copy(table_hbm.at[idx_v], rows_v, sem).wait()
        pltpu.sync_copy(rows_v, out_hbm.at[pl.ds(base, b_per_w)])

    return gather_kernel


@functools.lru_cache(maxsize=1)
def _make_sc_chunk_gather():
    info = plsc.get_sparse_core_info()
    nw = info.num_cores * info.num_subcores
    b_q = Q // nw
    mesh = plsc.VectorSubcoreMesh(core_axis_name="c", subcore_axis_name="s")

    @functools.partial(
        pl.kernel, mesh=mesh,
        out_type=jax.ShapeDtypeStruct((K5 * Q, CW), jnp.float32),
        scratch_types=[
            pltpu.VMEM((K5, b_q), jnp.int32),
            pltpu.VMEM((K5 * b_q, CW), jnp.float32),
            pltpu.SemaphoreType.DMA,
        ],
    )
    def gather_kernel(table_hbm, i0, i1, i2, i3, i4, out_hbm,
                      idx_v, rows_v, sem):
        wid = lax.axis_index("s") * info.num_cores + lax.axis_index("c")
        base = wid * b_q
        idx_hbms = [i0, i1, i2, i3, i4]
        for s in range(K5):
            pltpu.sync_copy(idx_hbms[s].at[pl.ds(base, b_q)], idx_v.at[s])
        copies = [
            pltpu.async_copy(table_hbm.at[idx_v.at[s]],
                             rows_v.at[pl.ds(s * b_q, b_q)], sem)
            for s in range(K5)
        ]
        for c in copies:
            c.wait()
        for s in range(K5):
            pltpu.sync_copy(rows_v.at[pl.ds(s * b_q, b_q)],
                            out_hbm.at[pl.ds(s * Q + base, b_q)])

    return gather_kernel


def _extract_topk(vals, ids, n):
    out_v, out_i = [], []
    for r in range(n):
        m = jnp.max(vals, axis=1, keepdims=True)
        sel = jnp.min(jnp.where(vals == m, ids, IMAX), axis=1, keepdims=True)
        out_v.append(m)
        out_i.append(sel)
        if r < n - 1:
            vals = jnp.where(ids == sel, -jnp.inf, vals)
    return jnp.concatenate(out_v, axis=1), jnp.concatenate(out_i, axis=1)


def _extract_topk_ax0(vals, ids, n):
    out_v, out_i = [], []
    for r in range(n):
        m = jnp.max(vals, axis=0, keepdims=True)
        sel = jnp.min(jnp.where(vals == m, ids, IMAX), axis=0, keepdims=True)
        out_v.append(m)
        out_i.append(sel)
        if r < n - 1:
            vals = jnp.where(ids == sel, -jnp.inf, vals)
    return jnp.concatenate(out_v, axis=0), jnp.concatenate(out_i, axis=0)


def _body_a(q_ref, emb_ref, sims_ref, base_ref,
            i0_ref, i1_ref, i2_ref, i3_ref, i4_ref, qn_ref, cmax_ref):
    v = pl.program_id(0)

    @pl.when(v == 0)
    def _init():
        q = q_ref[...]
        qn_ref[...] = q / jnp.sqrt(jnp.sum(q * q, axis=1, keepdims=True))

    emb = emb_ref[...]
    embn = emb / jnp.sqrt(jnp.sum(emb * emb, axis=1, keepdims=True))
    sims = lax.dot_general(qn_ref[...], embn, (((1,), (1,)), ((), ())),
                           preferred_element_type=jnp.float32)

    parts = []
    for s in range(CPT):
        slab = sims[:, s * CW:(s + 1) * CW]
        sims_ref[s] = slab
        parts.append(jnp.max(slab, axis=1, keepdims=True))
    cmax_ref[pl.ds(pl.multiple_of(v * CPT, CPT), CPT), :] = (
        jnp.concatenate(parts, axis=1).T)

    @pl.when(v == NT - 1)
    def _fix_tail_and_select():
        fparts = []
        if _REM:
            lane = lax.broadcasted_iota(jnp.int32, (Q, CW), 1)
            s_p = sims[:, _FULL_SLABS * CW:(_FULL_SLABS + 1) * CW]
            fparts.append(jnp.max(jnp.where(lane < _REM, s_p, -jnp.inf),
                                  axis=1, keepdims=True))
        while _FULL_SLABS + len(fparts) < CPT:
            fparts.append(jnp.full((Q, 1), -jnp.inf, jnp.float32))
        cmax_ref[pl.ds(NCH - len(fparts), len(fparts)), :] = (
            jnp.concatenate(fparts, axis=1).T)

        cids = lax.broadcasted_iota(jnp.int32, (NCH, Q), 0)
        _, top_c = _extract_topk_ax0(cmax_ref[...], cids, K5)
        base_ref[...] = (top_c * CW).T
        flat = top_c * Q + lax.broadcasted_iota(jnp.int32, (K5, Q), 1)
        for s, ref in enumerate([i0_ref, i1_ref, i2_ref, i3_ref, i4_ref]):
            ref[...] = flat[s]


_tc_a = pl.pallas_call(
    _body_a,
    grid=(NT,),
    in_specs=[
        pl.BlockSpec((Q, D), lambda v: (0, 0)),
        pl.BlockSpec((VT, D), lambda v: (v, 0)),
    ],
    out_specs=[
        pl.BlockSpec((CPT, Q, CW), lambda v: (v, 0, 0)),
        pl.BlockSpec((Q, K5), lambda v: (0, 0)),
    ] + [pl.BlockSpec((Q,), lambda v: (0,)) for _ in range(K5)],
    out_shape=[
        jax.ShapeDtypeStruct((NCH, Q, CW), jnp.float32),
        jax.ShapeDtypeStruct((Q, K5), jnp.int32),
    ] + [jax.ShapeDtypeStruct((Q,), jnp.int32) for _ in range(K5)],
    scratch_shapes=[
        pltpu.VMEM((Q, D), jnp.float32),
        pltpu.VMEM((NCH, Q), jnp.float32),
    ],
    compiler_params=pltpu.CompilerParams(
        dimension_semantics=("arbitrary",)),
)


def _body_b(k_ref, cand_ref, base_ref, outv_ref, outi_ref):
    cand = jnp.concatenate([cand_ref[s] for s in range(K5)], axis=1)
    cb = base_ref[...]
    iota = lax.broadcasted_iota(jnp.int32, (Q, CW), 1)
    ids = jnp.concatenate([cb[:, s:s + 1] + iota for s in range(K5)], axis=1)
    cand = jnp.where(ids < V, cand, -jnp.inf)
    top_v, top_i = _extract_topk(cand, ids, K5)
    outv_ref[...] = top_v
    outi_ref[...] = top_i + (k_ref[0] - K5)


_tc_b = pl.pallas_call(
    _body_b,
    in_specs=[
        pl.BlockSpec(memory_space=pltpu.SMEM),
        pl.BlockSpec(),
        pl.BlockSpec(),
    ],
    out_shape=[
        jax.ShapeDtypeStruct((Q, K5), jnp.float32),
        jax.ShapeDtypeStruct((Q, K5), jnp.int32),
    ],
)


def kernel(embedding, query_ids, k):
    q = _make_sc_query_gather()(embedding, query_ids.astype(jnp.int32))
    sims, col_base, i0, i1, i2, i3, i4 = _tc_a(q, embedding)
    table = sims.reshape(NCH * Q, CW)
    gathered = _make_sc_chunk_gather()(table, i0, i1, i2, i3, i4)
    cand3 = gathered.reshape(K5, Q, CW)
    k_arr = jnp.asarray(k, jnp.int32).reshape(1)
    top_v, top_i = _tc_b(k_arr, cand3, col_base)
    return top_v, top_i
